# baseline (speedup 1.0000x reference)
import functools

import jax
import jax.numpy as jnp
from jax import lax
from jax.experimental import pallas as pl
from jax.experimental.pallas import tpu as pltpu
from jax.experimental.pallas import tpu_sc as plsc

N = 10000
E = 320000
D = 128
HD = 64
C = 40
NC = 2
NS = 16
K = 80
WIN = 256
EPT = E // NS
PADE = WIN * K - EPT
ROWS_PER_TILE = 640
NP = NS * ROWS_PER_TILE
ZROWS = 64
GW = 4
NB = 2 * GW
RB = 1000
GRID = N // RB


def _deg_body(dst_hbm, out_hbm, dstv, ones_b, zbuf, accum, sem):
    c = lax.axis_index("c")
    s = lax.axis_index("s")
    nw = WIN // NC
    pltpu.sync_copy(dst_hbm.at[s], dstv)

    def fill_ones(i, _):
        ones_b[i] = jnp.ones((16,), jnp.float32)
        return _

    lax.fori_loop(0, K, fill_ones, None)

    def fill_zero(i, _):
        zbuf[i] = jnp.zeros((16,), jnp.float32)
        return _

    lax.fori_loop(0, ZROWS, fill_zero, None)
    for j in range(ROWS_PER_TILE // ZROWS):
        pltpu.sync_copy(zbuf, accum.at[pl.ds(ROWS_PER_TILE * s + ZROWS * j, ZROWS)])
    plsc.subcore_barrier()

    def fire(w, _):
        pltpu.make_async_copy(ones_b, accum.at[dstv.at[w]], sem).start(add=True)
        return _

    lax.fori_loop(nw * c, nw * (c + 1), fire, None)

    def drain(w, _):
        pltpu.make_async_copy(ones_b, accum.at[dstv.at[0]], sem).wait()
        return _

    lax.fori_loop(0, nw, drain, None)
    plsc.subcore_barrier()
    rows = pl.ds(ROWS_PER_TILE * s, ROWS_PER_TILE)
    pltpu.sync_copy(accum.at[rows], out_hbm.at[c, s])


@functools.cache
def _deg_call():
    mesh = plsc.VectorSubcoreMesh(
        core_axis_name="c", subcore_axis_name="s",
        num_cores=NC, num_subcores=NS)
    return pl.kernel(
        _deg_body,
        out_type=jax.ShapeDtypeStruct((NC, NS, ROWS_PER_TILE, 16), jnp.float32),
        mesh=mesh,
        scratch_types=[
            pltpu.VMEM((WIN, K), jnp.int32),
            pltpu.VMEM((K, 16), jnp.float32),
            pltpu.VMEM((ZROWS, 16), jnp.float32),
            pltpu.VMEM_SHARED((NP, 16), jnp.float32),
            pltpu.SemaphoreType.DMA,
        ],
        compiler_params=pltpu.CompilerParams(use_tc_tiling_on_sc=False),
    )


def _prop_body(hs_hbm, src_hbm, dst_hbm, out_hbm,
               srcv, dstv, b0, b1, b2, b3, b4, b5, b6, b7, zbuf, accum,
               g0, g1, g2, g3, g4, g5, g6, g7,
               s0, s1, s2, s3, s4, s5, s6, s7):
    c = lax.axis_index("c")
    s = lax.axis_index("s")
    pltpu.sync_copy(src_hbm.at[c, s], srcv)
    pltpu.sync_copy(dst_hbm.at[s], dstv)

    hd = zbuf.shape[1]

    def fill_zero(i, _):
        r = i // (hd // 16)
        k = (i % (hd // 16)) * 16
        zbuf[r, pl.ds(k, 16)] = jnp.zeros((16,), jnp.float32)
        return _

    lax.fori_loop(0, ZROWS * (hd // 16), fill_zero, None)

    def zfire(j, _):
        pltpu.make_async_copy(
            zbuf, accum.at[pl.ds(ROWS_PER_TILE * s + ZROWS * j, ZROWS)],
            g0).start()
        return _

    lax.fori_loop(0, ROWS_PER_TILE // ZROWS, zfire, None)

    def zdrain(j, _):
        pltpu.make_async_copy(
            zbuf, accum.at[pl.ds(ROWS_PER_TILE * s, ZROWS)], g0).wait()
        return _

    lax.fori_loop(0, ROWS_PER_TILE // ZROWS, zdrain, None)
    plsc.subcore_barrier()

    bufs = (b0, b1, b2, b3, b4, b5, b6, b7)
    gsems = (g0, g1, g2, g3, g4, g5, g6, g7)
    ssems = (s0, s1, s2, s3, s4, s5, s6, s7)

    def gstart(w, b):
        pltpu.make_async_copy(hs_hbm.at[srcv.at[w]], bufs[b], gsems[b]).start()

    def gwait(b):
        pltpu.make_async_copy(hs_hbm.at[srcv.at[0]], bufs[b], gsems[b]).wait()

    def sstart(w, b):
        pltpu.make_async_copy(bufs[b], accum.at[dstv.at[w]],
                              ssems[b]).start(add=True)

    def swait(b):
        pltpu.make_async_copy(bufs[b], accum.at[dstv.at[0]], ssems[b]).wait()

    def pair(j, first):
        for t in range(2):
            base = (2 * j + t) * GW
            for i in range(GW):
                b = GW * t + i
                if not first:
                    swait(b)
                gstart(base + i, b)
            for i in range(GW):
                b = GW * t + i
                gwait(b)
                sstart(base + i, b)

    pair(0, True)

    def body(j, _):
        pair(j, False)
        return _

    lax.fori_loop(1, WIN // (2 * GW), body, None)
    for b in range(NB):
        swait(b)
    plsc.subcore_barrier()
    rows = pl.ds(ROWS_PER_TILE * s, ROWS_PER_TILE)
    pltpu.sync_copy(accum.at[rows], out_hbm.at[c, s])


@functools.cache
def _prop_call(hd):
    mesh = plsc.VectorSubcoreMesh(
        core_axis_name="c", subcore_axis_name="s",
        num_cores=NC, num_subcores=NS)
    return pl.kernel(
        _prop_body,
        out_type=jax.ShapeDtypeStruct((NC, NS, ROWS_PER_TILE, hd), jnp.float32),
        mesh=mesh,
        scratch_types=(
            [pltpu.VMEM((WIN, K), jnp.int32)] * 2
            + [pltpu.VMEM((K, hd), jnp.float32)] * NB
            + [pltpu.VMEM((ZROWS, hd), jnp.float32),
               pltpu.VMEM_SHARED((NP, hd), jnp.float32)]
            + [pltpu.SemaphoreType.DMA] * (2 * NB)
        ),
        compiler_params=pltpu.CompilerParams(use_tc_tiling_on_sc=False),
    )


def _first_body(deg_ref, x_ref, w_ref, hs_ref, dinv_ref):
    deg = deg_ref[0] + deg_ref[1] + 1.0
    dinv = lax.rsqrt(deg)
    h = jnp.dot(x_ref[...], w_ref[...], preferred_element_type=jnp.float32)
    hs = h * dinv[:, :1]
    hs_ref[0] = hs[:, :HD]
    hs_ref[1] = hs[:, HD:]
    dinv_ref[...] = dinv


_first_call = pl.pallas_call(
    _first_body,
    grid=(GRID,),
    in_specs=[
        pl.BlockSpec((NC, RB, 16), lambda i: (0, i, 0)),
        pl.BlockSpec((RB, D), lambda i: (i, 0)),
        pl.BlockSpec((D, D), lambda i: (0, 0)),
    ],
    out_specs=[
        pl.BlockSpec((NC, RB, HD), lambda i: (0, i, 0)),
        pl.BlockSpec((RB, 16), lambda i: (i, 0)),
    ],
    out_shape=[
        jax.ShapeDtypeStruct((NC, N, HD), jnp.float32),
        jax.ShapeDtypeStruct((N, 16), jnp.float32),
    ],
)


def _mid_body(s_ref, hs_ref, dinv_ref, b_ref, w_ref, out_ref):
    dinv = dinv_ref[:, :1]
    z = jnp.concatenate(
        [s_ref[0] + hs_ref[0], s_ref[1] + hs_ref[1]], axis=1)
    z = z * dinv + b_ref[...][None, :]
    z = jnp.maximum(z, 0.0)
    h = jnp.dot(z, w_ref[...], preferred_element_type=jnp.float32)
    h = h * dinv
    hw = out_ref.shape[-1]
    out_ref[0] = h[:, :hw]
    out_ref[1] = h[:, hw:]


_mid_call = pl.pallas_call(
    _mid_body,
    grid=(GRID,),
    in_specs=[
        pl.BlockSpec((NC, RB, HD), lambda i: (0, i, 0)),
        pl.BlockSpec((NC, RB, HD), lambda i: (0, i, 0)),
        pl.BlockSpec((RB, 16), lambda i: (i, 0)),
        pl.BlockSpec((D,), lambda i: (0,)),
        pl.BlockSpec((D, D), lambda i: (0, 0)),
    ],
    out_specs=pl.BlockSpec((NC, RB, HD), lambda i: (0, i, 0)),
    out_shape=jax.ShapeDtypeStruct((NC, N, HD), jnp.float32),
)


HD4 = 32
D4 = 2 * HD4


_mid4_call = pl.pallas_call(
    _mid_body,
    grid=(GRID,),
    in_specs=[
        pl.BlockSpec((NC, RB, HD), lambda i: (0, i, 0)),
        pl.BlockSpec((NC, RB, HD), lambda i: (0, i, 0)),
        pl.BlockSpec((RB, 16), lambda i: (i, 0)),
        pl.BlockSpec((D,), lambda i: (0,)),
        pl.BlockSpec((D, D4), lambda i: (0, 0)),
    ],
    out_specs=pl.BlockSpec((NC, RB, HD4), lambda i: (0, i, 0)),
    out_shape=jax.ShapeDtypeStruct((NC, N, HD4), jnp.float32),
)


def _final_body(s_ref, hs_ref, dinv_ref, b_ref, out_ref):
    dinv = dinv_ref[:, :1]
    z = jnp.concatenate(
        [s_ref[0] + hs_ref[0], s_ref[1] + hs_ref[1]], axis=1)
    z = z * dinv
    logits = z[:, :C] + b_ref[...][None, :]
    m = jnp.max(logits, axis=1, keepdims=True)
    lse = jnp.log(jnp.sum(jnp.exp(logits - m), axis=1, keepdims=True)) + m
    out_ref[...] = logits - lse


_final_call = pl.pallas_call(
    _final_body,
    grid=(GRID,),
    in_specs=[
        pl.BlockSpec((NC, RB, HD4), lambda i: (0, i, 0)),
        pl.BlockSpec((NC, RB, HD4), lambda i: (0, i, 0)),
        pl.BlockSpec((RB, 16), lambda i: (i, 0)),
        pl.BlockSpec((C,), lambda i: (0,)),
    ],
    out_specs=pl.BlockSpec((RB, C), lambda i: (i, 0)),
    out_shape=jax.ShapeDtypeStruct((N, C), jnp.float32),
)


@jax.jit
def kernel(x, edge_index, W1, b1, W2, b2, W3, b3, W4, b4):
    pad_src = jnp.broadcast_to(
        (jnp.arange(PADE, dtype=jnp.int32) * 61) % N, (NS, PADE))
    pad_dst = jnp.broadcast_to(
        N + (jnp.arange(PADE, dtype=jnp.int32) * 13) % (NP - N), (NS, PADE))
    src = jnp.concatenate(
        [edge_index[0].astype(jnp.int32).reshape(NS, EPT), pad_src],
        axis=1).reshape(NS, WIN, K)
    dst = jnp.concatenate(
        [edge_index[1].astype(jnp.int32).reshape(NS, EPT), pad_dst],
        axis=1).reshape(NS, WIN, K)
    src2 = jnp.stack([src, src + N])

    deg16 = _deg_call()(dst).reshape(NC, NP, 16)
    hs, dinv16 = _first_call(deg16, x, W1)

    for b, w_next in ((b1, W2), (b2, W3)):
        s = _prop_call(HD)(hs.reshape(NC * N, HD), src2, dst).reshape(NC, NP, HD)
        hs = _mid_call(s, hs, dinv16, b, w_next)

    w4p = jnp.pad(W4, ((0, 0), (0, D4 - C)))
    s3 = _prop_call(HD)(hs.reshape(NC * N, HD), src2, dst).reshape(NC, NP, HD)
    hs4 = _mid4_call(s3, hs, dinv16, b3, w4p)
    s4 = _prop_call(HD4)(hs4.reshape(NC * N, HD4), src2, dst).reshape(NC, NP, HD4)
    return _final_call(s4, hs4, dinv16, b4)


# speedup vs baseline: 1.0157x; 1.0157x over previous
import functools

import jax
import jax.numpy as jnp
from jax import lax
from jax.experimental import pallas as pl
from jax.experimental.pallas import tpu as pltpu
from jax.experimental.pallas import tpu_sc as plsc

N = 10000
E = 320000
D = 128
HD = 64
C = 40
NC = 2
NS = 16
K = 80
WIN = 256
EPT = E // NS
PADE = WIN * K - EPT
ROWS_PER_TILE = 640
NP = NS * ROWS_PER_TILE
ZROWS = 64
GW = 4
NB = 2 * GW
RB = 2000
GRID = N // RB


def _deg_body(dst_hbm, out_hbm, dstv, ones_b, zbuf, accum, sem):
    c = lax.axis_index("c")
    s = lax.axis_index("s")
    nw = WIN // NC
    pltpu.sync_copy(dst_hbm.at[s], dstv)

    def fill_ones(i, _):
        ones_b[i] = jnp.ones((16,), jnp.float32)
        return _

    lax.fori_loop(0, K, fill_ones, None)

    def fill_zero(i, _):
        zbuf[i] = jnp.zeros((16,), jnp.float32)
        return _

    lax.fori_loop(0, ZROWS, fill_zero, None)
    for j in range(ROWS_PER_TILE // ZROWS):
        pltpu.sync_copy(zbuf, accum.at[pl.ds(ROWS_PER_TILE * s + ZROWS * j, ZROWS)])
    plsc.subcore_barrier()

    def fire(w, _):
        pltpu.make_async_copy(ones_b, accum.at[dstv.at[w]], sem).start(add=True)
        return _

    lax.fori_loop(nw * c, nw * (c + 1), fire, None)

    def drain(w, _):
        pltpu.make_async_copy(ones_b, accum.at[dstv.at[0]], sem).wait()
        return _

    lax.fori_loop(0, nw, drain, None)
    plsc.subcore_barrier()
    rows = pl.ds(ROWS_PER_TILE * s, ROWS_PER_TILE)
    pltpu.sync_copy(accum.at[rows], out_hbm.at[c, s])


@functools.cache
def _deg_call():
    mesh = plsc.VectorSubcoreMesh(
        core_axis_name="c", subcore_axis_name="s",
        num_cores=NC, num_subcores=NS)
    return pl.kernel(
        _deg_body,
        out_type=jax.ShapeDtypeStruct((NC, NS, ROWS_PER_TILE, 16), jnp.float32),
        mesh=mesh,
        scratch_types=[
            pltpu.VMEM((WIN, K), jnp.int32),
            pltpu.VMEM((K, 16), jnp.float32),
            pltpu.VMEM((ZROWS, 16), jnp.float32),
            pltpu.VMEM_SHARED((NP, 16), jnp.float32),
            pltpu.SemaphoreType.DMA,
        ],
        compiler_params=pltpu.CompilerParams(use_tc_tiling_on_sc=False),
    )


def _prop_body(hs_hbm, src_hbm, dst_hbm, out_hbm,
               srcv, dstv, b0, b1, b2, b3, b4, b5, b6, b7, zbuf, accum,
               g0, g1, g2, g3, g4, g5, g6, g7,
               s0, s1, s2, s3, s4, s5, s6, s7):
    c = lax.axis_index("c")
    s = lax.axis_index("s")
    pltpu.sync_copy(src_hbm.at[c, s], srcv)
    pltpu.sync_copy(dst_hbm.at[s], dstv)

    hd = zbuf.shape[1]

    def fill_zero(i, _):
        r = i // (hd // 16)
        k = (i % (hd // 16)) * 16
        zbuf[r, pl.ds(k, 16)] = jnp.zeros((16,), jnp.float32)
        return _

    lax.fori_loop(0, ZROWS * (hd // 16), fill_zero, None)

    def zfire(j, _):
        pltpu.make_async_copy(
            zbuf, accum.at[pl.ds(ROWS_PER_TILE * s + ZROWS * j, ZROWS)],
            g0).start()
        return _

    lax.fori_loop(0, ROWS_PER_TILE // ZROWS, zfire, None)

    def zdrain(j, _):
        pltpu.make_async_copy(
            zbuf, accum.at[pl.ds(ROWS_PER_TILE * s, ZROWS)], g0).wait()
        return _

    lax.fori_loop(0, ROWS_PER_TILE // ZROWS, zdrain, None)
    plsc.subcore_barrier()

    bufs = (b0, b1, b2, b3, b4, b5, b6, b7)
    gsems = (g0, g1, g2, g3, g4, g5, g6, g7)
    ssems = (s0, s1, s2, s3, s4, s5, s6, s7)

    def gstart(w, b):
        pltpu.make_async_copy(hs_hbm.at[srcv.at[w]], bufs[b], gsems[b]).start()

    def gwait(b):
        pltpu.make_async_copy(hs_hbm.at[srcv.at[0]], bufs[b], gsems[b]).wait()

    def sstart(w, b):
        pltpu.make_async_copy(bufs[b], accum.at[dstv.at[w]],
                              ssems[b]).start(add=True)

    def swait(b):
        pltpu.make_async_copy(bufs[b], accum.at[dstv.at[0]], ssems[b]).wait()

    def pair(j, first):
        for t in range(2):
            base = (2 * j + t) * GW
            for i in range(GW):
                b = GW * t + i
                if not first:
                    swait(b)
                gstart(base + i, b)
            for i in range(GW):
                b = GW * t + i
                gwait(b)
                sstart(base + i, b)

    pair(0, True)

    def body(j, _):
        pair(j, False)
        return _

    lax.fori_loop(1, WIN // (2 * GW), body, None)
    for b in range(NB):
        swait(b)
    plsc.subcore_barrier()
    rows = pl.ds(ROWS_PER_TILE * s, ROWS_PER_TILE)
    pltpu.sync_copy(accum.at[rows], out_hbm.at[c, s])


@functools.cache
def _prop_call(hd):
    mesh = plsc.VectorSubcoreMesh(
        core_axis_name="c", subcore_axis_name="s",
        num_cores=NC, num_subcores=NS)
    return pl.kernel(
        _prop_body,
        out_type=jax.ShapeDtypeStruct((NC, NS, ROWS_PER_TILE, hd), jnp.float32),
        mesh=mesh,
        scratch_types=(
            [pltpu.VMEM((WIN, K), jnp.int32)] * 2
            + [pltpu.VMEM((K, hd), jnp.float32)] * NB
            + [pltpu.VMEM((ZROWS, hd), jnp.float32),
               pltpu.VMEM_SHARED((NP, hd), jnp.float32)]
            + [pltpu.SemaphoreType.DMA] * (2 * NB)
        ),
        compiler_params=pltpu.CompilerParams(use_tc_tiling_on_sc=False),
    )


def _first_body(deg_ref, x_ref, w_ref, hs_ref, dinv_ref):
    deg = deg_ref[0] + deg_ref[1] + 1.0
    dinv = lax.rsqrt(deg)
    h = jnp.dot(x_ref[...], w_ref[...], preferred_element_type=jnp.float32)
    hs = h * dinv[:, :1]
    hs_ref[0] = hs[:, :HD]
    hs_ref[1] = hs[:, HD:]
    dinv_ref[...] = dinv


_first_call = pl.pallas_call(
    _first_body,
    grid=(GRID,),
    in_specs=[
        pl.BlockSpec((NC, RB, 16), lambda i: (0, i, 0)),
        pl.BlockSpec((RB, D), lambda i: (i, 0)),
        pl.BlockSpec((D, D), lambda i: (0, 0)),
    ],
    out_specs=[
        pl.BlockSpec((NC, RB, HD), lambda i: (0, i, 0)),
        pl.BlockSpec((RB, 16), lambda i: (i, 0)),
    ],
    out_shape=[
        jax.ShapeDtypeStruct((NC, N, HD), jnp.float32),
        jax.ShapeDtypeStruct((N, 16), jnp.float32),
    ],
)


def _mid_body(s_ref, hs_ref, dinv_ref, b_ref, w_ref, out_ref):
    dinv = dinv_ref[:, :1]
    z = jnp.concatenate(
        [s_ref[0] + hs_ref[0], s_ref[1] + hs_ref[1]], axis=1)
    z = z * dinv + b_ref[...][None, :]
    z = jnp.maximum(z, 0.0)
    h = jnp.dot(z, w_ref[...], preferred_element_type=jnp.float32)
    h = h * dinv
    hw = out_ref.shape[-1]
    out_ref[0] = h[:, :hw]
    out_ref[1] = h[:, hw:]


_mid_call = pl.pallas_call(
    _mid_body,
    grid=(GRID,),
    in_specs=[
        pl.BlockSpec((NC, RB, HD), lambda i: (0, i, 0)),
        pl.BlockSpec((NC, RB, HD), lambda i: (0, i, 0)),
        pl.BlockSpec((RB, 16), lambda i: (i, 0)),
        pl.BlockSpec((D,), lambda i: (0,)),
        pl.BlockSpec((D, D), lambda i: (0, 0)),
    ],
    out_specs=pl.BlockSpec((NC, RB, HD), lambda i: (0, i, 0)),
    out_shape=jax.ShapeDtypeStruct((NC, N, HD), jnp.float32),
)


HD4 = 32
D4 = 2 * HD4


_mid4_call = pl.pallas_call(
    _mid_body,
    grid=(GRID,),
    in_specs=[
        pl.BlockSpec((NC, RB, HD), lambda i: (0, i, 0)),
        pl.BlockSpec((NC, RB, HD), lambda i: (0, i, 0)),
        pl.BlockSpec((RB, 16), lambda i: (i, 0)),
        pl.BlockSpec((D,), lambda i: (0,)),
        pl.BlockSpec((D, D4), lambda i: (0, 0)),
    ],
    out_specs=pl.BlockSpec((NC, RB, HD4), lambda i: (0, i, 0)),
    out_shape=jax.ShapeDtypeStruct((NC, N, HD4), jnp.float32),
)


def _final_body(s_ref, hs_ref, dinv_ref, b_ref, out_ref):
    dinv = dinv_ref[:, :1]
    z = jnp.concatenate(
        [s_ref[0] + hs_ref[0], s_ref[1] + hs_ref[1]], axis=1)
    z = z * dinv
    logits = z[:, :C] + b_ref[...][None, :]
    m = jnp.max(logits, axis=1, keepdims=True)
    lse = jnp.log(jnp.sum(jnp.exp(logits - m), axis=1, keepdims=True)) + m
    out_ref[...] = logits - lse


_final_call = pl.pallas_call(
    _final_body,
    grid=(GRID,),
    in_specs=[
        pl.BlockSpec((NC, RB, HD4), lambda i: (0, i, 0)),
        pl.BlockSpec((NC, RB, HD4), lambda i: (0, i, 0)),
        pl.BlockSpec((RB, 16), lambda i: (i, 0)),
        pl.BlockSpec((C,), lambda i: (0,)),
    ],
    out_specs=pl.BlockSpec((RB, C), lambda i: (i, 0)),
    out_shape=jax.ShapeDtypeStruct((N, C), jnp.float32),
)


@jax.jit
def kernel(x, edge_index, W1, b1, W2, b2, W3, b3, W4, b4):
    pad_src = jnp.broadcast_to(
        (jnp.arange(PADE, dtype=jnp.int32) * 61) % N, (NS, PADE))
    pad_dst = jnp.broadcast_to(
        N + (jnp.arange(PADE, dtype=jnp.int32) * 13) % (NP - N), (NS, PADE))
    src = jnp.concatenate(
        [edge_index[0].astype(jnp.int32).reshape(NS, EPT), pad_src],
        axis=1).reshape(NS, WIN, K)
    dst = jnp.concatenate(
        [edge_index[1].astype(jnp.int32).reshape(NS, EPT), pad_dst],
        axis=1).reshape(NS, WIN, K)
    src2 = jnp.stack([src, src + N])

    deg16 = _deg_call()(dst).reshape(NC, NP, 16)
    hs, dinv16 = _first_call(deg16, x, W1)

    for b, w_next in ((b1, W2), (b2, W3)):
        s = _prop_call(HD)(hs.reshape(NC * N, HD), src2, dst).reshape(NC, NP, HD)
        hs = _mid_call(s, hs, dinv16, b, w_next)

    w4p = jnp.pad(W4, ((0, 0), (0, D4 - C)))
    s3 = _prop_call(HD)(hs.reshape(NC * N, HD), src2, dst).reshape(NC, NP, HD)
    hs4 = _mid4_call(s3, hs, dinv16, b3, w4p)
    s4 = _prop_call(HD4)(hs4.reshape(NC * N, HD4), src2, dst).reshape(NC, NP, HD4)
    return _final_call(s4, hs4, dinv16, b4)
